# baseline (speedup 1.0000x reference)
import functools

import jax
import jax.numpy as jnp
from jax import lax
from jax.experimental import pallas as pl
from jax.experimental.pallas import tpu as pltpu
from jax.experimental.pallas import tpu_sc as plsc

B = 16384
V = 1000000
D = 32
L = 16
NC = 2
NS = 16
NW = NC * NS
BPW = B // NW
CHUNK = 128
NCHUNK = BPW // CHUNK
PACK = 128 // D
RBLK = 2048
NPACK = V // PACK
_GRID = (V + RBLK - 1) // RBLK


_NGRP = (V + 511) // 512
_GPW = (_NGRP + NW - 1) // NW

_mesh_rp = plsc.VectorSubcoreMesh(core_axis_name="c", subcore_axis_name="s")


@functools.partial(
    pl.kernel,
    out_type=jax.ShapeDtypeStruct((_NGRP * 128, PACK * D), jnp.float32),
    mesh=_mesh_rp,
    compiler_params=pltpu.CompilerParams(
        needs_layout_passes=False, use_tc_tiling_on_sc=True),
    scratch_types=[
        pltpu.VMEM((D, 512), jnp.float32),
        pltpu.VMEM((128, PACK * D), jnp.float32),
        pltpu.VMEM((D, 64), jnp.float32),
    ],
)
def _repack(tabT_hbm, tail_hbm, out_hbm, win_v, blk_v, tl_v):
    wid = lax.axis_index("s") * NC + lax.axis_index("c")
    base = wid * _GPW
    n_own = jnp.minimum(_GPW, jnp.maximum((_NGRP - 1) - base, 0))
    lanes = lax.iota(jnp.int32, L)

    def group_body(i_loc, carry):
        i = base + i_loc
        pltpu.sync_copy(tabT_hbm.at[:, pl.ds(i * 512, 512)], win_v)
        for d in range(D):
            for s in range(PACK):
                col = jnp.full((L,), s * D + d, jnp.int32)
                for k in range(128 // L):
                    xv = k * L + lanes
                    v = win_v[d, pl.ds(s * 128 + k * L, L)]
                    plsc.store_scatter(blk_v, [xv, col], v)
        pltpu.sync_copy(blk_v, out_hbm.at[pl.ds(i * 128, 128)])
        return carry

    lax.fori_loop(0, n_own, group_body, 0)

    @pl.when(wid == NW - 1)
    def _tail():
        pltpu.sync_copy(tail_hbm, tl_v)
        for d in range(D):
            col = jnp.full((L,), d, jnp.int32)
            for k in range(64 // L):
                xv = k * L + lanes
                v = tl_v[d, pl.ds(k * L, L)]
                plsc.store_scatter(blk_v, [xv, col], v)
        pltpu.sync_copy(blk_v, out_hbm.at[pl.ds((_NGRP - 1) * 128, 128)])


_mesh = plsc.VectorSubcoreMesh(core_axis_name="c", subcore_axis_name="s")


@functools.partial(
    pl.kernel,
    out_type=jax.ShapeDtypeStruct((B,), jnp.float32),
    mesh=_mesh,
    compiler_params=pltpu.CompilerParams(
        needs_layout_passes=False, use_tc_tiling_on_sc=False),
    scratch_types=[
        pltpu.VMEM((NCHUNK, CHUNK), jnp.int32),
        pltpu.VMEM((NCHUNK, CHUNK), jnp.int32),
        pltpu.VMEM((NCHUNK, CHUNK), jnp.int32),
        pltpu.VMEM((NCHUNK, CHUNK), jnp.int32),
        pltpu.VMEM((CHUNK, PACK * D), jnp.float32),
        pltpu.VMEM((CHUNK, PACK * D), jnp.float32),
        pltpu.VMEM((BPW,), jnp.float32),
        pltpu.SemaphoreType.DMA,
    ],
)
def _cf_kernel(user_hbm, item_hbm, upack_hbm, ipack_hbm, out_hbm,
               uidx_v, iidx_v, uq_v, iq_v, ubuf_v, ibuf_v, out_v, sem):
    wid = lax.axis_index("s") * NC + lax.axis_index("c")
    base = wid * BPW

    pltpu.sync_copy(user_hbm.at[pl.ds(wid * NCHUNK, NCHUNK)], uidx_v)
    pltpu.sync_copy(item_hbm.at[pl.ds(wid * NCHUNK, NCHUNK)], iidx_v)

    def _q(r):
        return (lax.shift_left(lax.shift_right_logical(r, 9), 7)
                + (r & 127))

    def qbody(v, carry):
        j = v // (CHUNK // L)
        col = (v % (CHUNK // L)) * L
        uq_v[j, pl.ds(col, L)] = _q(uidx_v[j, pl.ds(col, L)])
        iq_v[j, pl.ds(col, L)] = _q(iidx_v[j, pl.ds(col, L)])
        return carry

    lax.fori_loop(0, NCHUNK * CHUNK // L, qbody, 0)

    lanes = lax.iota(jnp.int32, L)

    def chunk_body(j, carry):
        cu = pltpu.async_copy(upack_hbm.at[uq_v.at[j]], ubuf_v, sem)
        ci = pltpu.async_copy(ipack_hbm.at[iq_v.at[j]], ibuf_v, sem)
        cu.wait()
        ci.wait()
        for k in range(CHUNK // L):
            pvec = k * L + lanes
            ubase = lax.shift_left(
                lax.shift_right_logical(uidx_v[j, pl.ds(k * L, L)], 7) & 3, 5)
            ibase = lax.shift_left(
                lax.shift_right_logical(iidx_v[j, pl.ds(k * L, L)], 7) & 3, 5)
            acc = jnp.zeros((L,), jnp.float32)
            for d in range(D):
                u = plsc.load_gather(ubuf_v, [pvec, ubase + d])
                it = plsc.load_gather(ibuf_v, [pvec, ibase + d])
                acc = acc + u * it
            out_v[pl.ds(j * CHUNK + k * L, L)] = acc
        return carry

    lax.fori_loop(0, NCHUNK, chunk_body, 0)

    pltpu.sync_copy(out_v, out_hbm.at[pl.ds(base, BPW)])


def kernel(user, item, user_table, item_table):
    utabT = user_table.T
    itabT = item_table.T
    upack = _repack(utabT, utabT[:, V - 64:])
    ipack = _repack(itabT, itabT[:, V - 64:])
    user2 = user.reshape(NW * NCHUNK, CHUNK)
    item2 = item.reshape(NW * NCHUNK, CHUNK)
    return _cf_kernel(user2, item2, upack, ipack)


# speedup vs baseline: 1.5095x; 1.5095x over previous
import functools

import jax
import jax.numpy as jnp
from jax import lax
from jax.experimental import pallas as pl
from jax.experimental.pallas import tpu as pltpu
from jax.experimental.pallas import tpu_sc as plsc

B = 16384
V = 1000000
D = 32
L = 16
NC = 2
NS = 16
NW = NC * NS
BPW = B // NW
CHUNK = 128
NCHUNK = BPW // CHUNK
PACK = 128 // D
RBLK = 2048
NPACK = V // PACK
_GRID = (V + RBLK - 1) // RBLK


_NGRP = (V + 511) // 512
_GPW = (_NGRP + NW - 1) // NW

_mesh_rp = plsc.VectorSubcoreMesh(core_axis_name="c", subcore_axis_name="s")


@functools.partial(
    pl.kernel,
    out_type=jax.ShapeDtypeStruct((_NGRP * 128, PACK * D), jnp.float32),
    mesh=_mesh_rp,
    compiler_params=pltpu.CompilerParams(
        needs_layout_passes=False, use_tc_tiling_on_sc=True),
    scratch_types=[
        pltpu.VMEM((D, 512), jnp.float32),
        pltpu.VMEM((128, PACK * D), jnp.float32),
        pltpu.VMEM((D, 64), jnp.float32),
    ],
)
def _repack(tabT_hbm, tail_hbm, out_hbm, win_v, blk_v, tl_v):
    wid = lax.axis_index("s") * NC + lax.axis_index("c")
    base = wid * _GPW
    n_own = jnp.minimum(_GPW, jnp.maximum((_NGRP - 1) - base, 0))
    lanes = lax.iota(jnp.int32, L)

    def group_body(i_loc, carry):
        i = base + i_loc
        pltpu.sync_copy(tabT_hbm.at[:, pl.ds(i * 512, 512)], win_v)

        def diag_body(j, carry2):
            perm = (lanes + j) & (L - 1)
            for d0 in range(0, D, L):
                dv = d0 + lanes
                for s in range(PACK):
                    for x0 in range(0, 128, L):
                        v = plsc.load_gather(
                            win_v, [dv, (s * 128 + x0) + perm])
                        plsc.store_scatter(
                            blk_v, [x0 + perm, (s * D + d0) + lanes], v)
            return carry2

        lax.fori_loop(0, L, diag_body, 0)
        pltpu.sync_copy(blk_v, out_hbm.at[pl.ds(i * 128, 128)])
        return carry

    lax.fori_loop(0, n_own, group_body, 0)

    @pl.when(wid == NW - 1)
    def _tail():
        pltpu.sync_copy(tail_hbm, tl_v)
        for d in range(D):
            col = jnp.full((L,), d, jnp.int32)
            for k in range(64 // L):
                xv = k * L + lanes
                v = tl_v[d, pl.ds(k * L, L)]
                plsc.store_scatter(blk_v, [xv, col], v)
        pltpu.sync_copy(blk_v, out_hbm.at[pl.ds((_NGRP - 1) * 128, 128)])


_mesh = plsc.VectorSubcoreMesh(core_axis_name="c", subcore_axis_name="s")


@functools.partial(
    pl.kernel,
    out_type=jax.ShapeDtypeStruct((B,), jnp.float32),
    mesh=_mesh,
    compiler_params=pltpu.CompilerParams(
        needs_layout_passes=False, use_tc_tiling_on_sc=False),
    scratch_types=[
        pltpu.VMEM((NCHUNK, CHUNK), jnp.int32),
        pltpu.VMEM((NCHUNK, CHUNK), jnp.int32),
        pltpu.VMEM((NCHUNK, CHUNK), jnp.int32),
        pltpu.VMEM((NCHUNK, CHUNK), jnp.int32),
        pltpu.VMEM((CHUNK, PACK * D), jnp.float32),
        pltpu.VMEM((CHUNK, PACK * D), jnp.float32),
        pltpu.VMEM((BPW,), jnp.float32),
        pltpu.SemaphoreType.DMA,
    ],
)
def _cf_kernel(user_hbm, item_hbm, upack_hbm, ipack_hbm, out_hbm,
               uidx_v, iidx_v, uq_v, iq_v, ubuf_v, ibuf_v, out_v, sem):
    wid = lax.axis_index("s") * NC + lax.axis_index("c")
    base = wid * BPW

    pltpu.sync_copy(user_hbm.at[pl.ds(wid * NCHUNK, NCHUNK)], uidx_v)
    pltpu.sync_copy(item_hbm.at[pl.ds(wid * NCHUNK, NCHUNK)], iidx_v)

    def _q(r):
        return (lax.shift_left(lax.shift_right_logical(r, 9), 7)
                + (r & 127))

    def qbody(v, carry):
        j = v // (CHUNK // L)
        col = (v % (CHUNK // L)) * L
        uq_v[j, pl.ds(col, L)] = _q(uidx_v[j, pl.ds(col, L)])
        iq_v[j, pl.ds(col, L)] = _q(iidx_v[j, pl.ds(col, L)])
        return carry

    lax.fori_loop(0, NCHUNK * CHUNK // L, qbody, 0)

    lanes = lax.iota(jnp.int32, L)

    def chunk_body(j, carry):
        cu = pltpu.async_copy(upack_hbm.at[uq_v.at[j]], ubuf_v, sem)
        ci = pltpu.async_copy(ipack_hbm.at[iq_v.at[j]], ibuf_v, sem)
        cu.wait()
        ci.wait()
        for k in range(CHUNK // L):
            pvec = k * L + lanes
            ubase = lax.shift_left(
                lax.shift_right_logical(uidx_v[j, pl.ds(k * L, L)], 7) & 3, 5)
            ibase = lax.shift_left(
                lax.shift_right_logical(iidx_v[j, pl.ds(k * L, L)], 7) & 3, 5)
            acc = jnp.zeros((L,), jnp.float32)
            for d in range(D):
                u = plsc.load_gather(ubuf_v, [pvec, ubase + d])
                it = plsc.load_gather(ibuf_v, [pvec, ibase + d])
                acc = acc + u * it
            out_v[pl.ds(j * CHUNK + k * L, L)] = acc
        return carry

    lax.fori_loop(0, NCHUNK, chunk_body, 0)

    pltpu.sync_copy(out_v, out_hbm.at[pl.ds(base, BPW)])


def kernel(user, item, user_table, item_table):
    utabT = user_table.T
    itabT = item_table.T
    upack = _repack(utabT, utabT[:, V - 64:])
    ipack = _repack(itabT, itabT[:, V - 64:])
    user2 = user.reshape(NW * NCHUNK, CHUNK)
    item2 = item.reshape(NW * NCHUNK, CHUNK)
    return _cf_kernel(user2, item2, upack, ipack)


# speedup vs baseline: 1.7256x; 1.1431x over previous
import functools

import jax
import jax.numpy as jnp
from jax import lax
from jax.experimental import pallas as pl
from jax.experimental.pallas import tpu as pltpu
from jax.experimental.pallas import tpu_sc as plsc

B = 16384
D = 32
L = 16
NC = 2
NS = 16
NW = NC * NS
BPW = B // NW
CHUNK = 128
NCHUNK = BPW // CHUNK

_mesh = plsc.VectorSubcoreMesh(core_axis_name="c", subcore_axis_name="s")


@functools.partial(
    pl.kernel,
    out_type=jax.ShapeDtypeStruct((B,), jnp.float32),
    mesh=_mesh,
    compiler_params=pltpu.CompilerParams(
        needs_layout_passes=False, use_tc_tiling_on_sc=False),
    scratch_types=[
        pltpu.VMEM((NCHUNK, CHUNK), jnp.int32),
        pltpu.VMEM((NCHUNK, CHUNK), jnp.int32),
        pltpu.VMEM((BPW, D), jnp.float32),
        pltpu.VMEM((BPW, D), jnp.float32),
        pltpu.VMEM((BPW,), jnp.float32),
        pltpu.SemaphoreType.DMA,
    ],
)
def _cf_kernel(user_hbm, item_hbm, utab_hbm, itab_hbm, out_hbm,
               uidx_v, iidx_v, urows_v, irows_v, out_v, sem):
    wid = lax.axis_index("s") * NC + lax.axis_index("c")
    base = wid * BPW

    pltpu.sync_copy(user_hbm.at[pl.ds(wid * NCHUNK, NCHUNK)], uidx_v)
    pltpu.sync_copy(item_hbm.at[pl.ds(wid * NCHUNK, NCHUNK)], iidx_v)

    copies = []
    for j in range(NCHUNK):
        copies.append(pltpu.async_copy(
            utab_hbm.at[uidx_v.at[j]],
            urows_v.at[pl.ds(j * CHUNK, CHUNK)], sem))
        copies.append(pltpu.async_copy(
            itab_hbm.at[iidx_v.at[j]],
            irows_v.at[pl.ds(j * CHUNK, CHUNK)], sem))
    for cp in copies:
        cp.wait()

    lanes = lax.iota(jnp.int32, L)

    def block_body(blk, carry):
        rows = blk * L + lanes
        acc = jnp.zeros((L,), jnp.float32)
        for d in range(D):
            col = jnp.full((L,), d, jnp.int32)
            u = plsc.load_gather(urows_v, [rows, col])
            it = plsc.load_gather(irows_v, [rows, col])
            acc = acc + u * it
        out_v[pl.ds(blk * L, L)] = acc
        return carry

    lax.fori_loop(0, BPW // L, block_body, 0)

    pltpu.sync_copy(out_v, out_hbm.at[pl.ds(base, BPW)])


def kernel(user, item, user_table, item_table):
    user2 = user.reshape(NW * NCHUNK, CHUNK)
    item2 = item.reshape(NW * NCHUNK, CHUNK)
    return _cf_kernel(user2, item2, user_table, item_table)


# speedup vs baseline: 2.8394x; 1.6455x over previous
import functools

import jax
import jax.numpy as jnp
from jax import lax
from jax.experimental import pallas as pl
from jax.experimental.pallas import tpu as pltpu
from jax.experimental.pallas import tpu_sc as plsc

B = 16384
V = 1000000
D = 32
L = 16
NC = 2
NS = 16
NW = NC * NS
BPW = B // NW
CHUNK = 128
NCHUNK = BPW // CHUNK
PACK = 128 // D
_NGRP = (V + 511) // 512
_GPW = (_NGRP + NW - 1) // NW
NPROWS = _NGRP * 128


RBLK = 2048
_GRID = (V + RBLK - 1) // RBLK


def _repack_tc_body(t_ref, o_ref):
    t = t_ref[...]
    tt = lax.dot_general(t, jnp.eye(D, dtype=jnp.float32),
                         (((0,), (0,)), ((), ())),
                         preferred_element_type=jnp.float32)
    for g in range(RBLK // 512):
        for s in range(PACK):
            lo = 512 * g + 128 * s
            o_ref[pl.ds(128 * g, 128), pl.ds(D * s, D)] = tt[lo:lo + 128, :]


_repack_tc = pl.pallas_call(
    _repack_tc_body,
    grid=(_GRID,),
    in_specs=[pl.BlockSpec((D, RBLK), lambda i: (0, i))],
    out_specs=pl.BlockSpec((RBLK // PACK, PACK * D), lambda i: (i, 0)),
    out_shape=jax.ShapeDtypeStruct((_GRID * (RBLK // PACK), PACK * D),
                                   jnp.float32),
    compiler_params=pltpu.CompilerParams(fuse_transposed_lhs_in_matmul=True),
)


_mesh_rp = plsc.VectorSubcoreMesh(core_axis_name="c", subcore_axis_name="s")


@functools.partial(
    pl.kernel,
    out_type=jax.ShapeDtypeStruct((NPROWS, PACK * D), jnp.float32),
    mesh=_mesh_rp,
    compiler_params=pltpu.CompilerParams(
        needs_layout_passes=False, use_tc_tiling_on_sc=True),
    scratch_types=[
        pltpu.VMEM((D, 512), jnp.float32),
        pltpu.VMEM((128, PACK * D), jnp.float32),
        pltpu.VMEM((D, 64), jnp.float32),
    ],
)
def _repack_sc(tabT_hbm, tail_hbm, out_hbm, win_v, blk_v, tl_v):
    wid = lax.axis_index("s") * NC + lax.axis_index("c")
    base = wid * _GPW
    n_own = jnp.minimum(_GPW, jnp.maximum((_NGRP - 1) - base, 0))
    lanes = lax.iota(jnp.int32, L)

    def group_body(i_loc, carry):
        i = base + i_loc
        pltpu.sync_copy(tabT_hbm.at[:, pl.ds(i * 512, 512)], win_v)

        def diag_body(j, carry2):
            perm = (lanes + j) & (L - 1)
            for d0 in range(0, D, L):
                dv = d0 + lanes
                for s in range(PACK):
                    for x0 in range(0, 128, L):
                        v = plsc.load_gather(
                            win_v, [dv, (s * 128 + x0) + perm])
                        plsc.store_scatter(
                            blk_v, [x0 + perm, (s * D + d0) + lanes], v)
            return carry2

        lax.fori_loop(0, L, diag_body, 0)
        pltpu.sync_copy(blk_v, out_hbm.at[pl.ds(i * 128, 128)])
        return carry

    lax.fori_loop(0, n_own, group_body, 0)

    @pl.when(wid == NW - 1)
    def _tail():
        pltpu.sync_copy(tail_hbm, tl_v)
        for d in range(D):
            col = jnp.full((L,), d, jnp.int32)
            for k in range(64 // L):
                xv = k * L + lanes
                v = tl_v[d, pl.ds(k * L, L)]
                plsc.store_scatter(blk_v, [xv, col], v)
        pltpu.sync_copy(blk_v, out_hbm.at[pl.ds((_NGRP - 1) * 128, 128)])


_mesh = plsc.VectorSubcoreMesh(core_axis_name="c", subcore_axis_name="s")


@functools.partial(
    pl.kernel,
    out_type=jax.ShapeDtypeStruct((B,), jnp.float32),
    mesh=_mesh,
    compiler_params=pltpu.CompilerParams(
        needs_layout_passes=False, use_tc_tiling_on_sc=False),
    scratch_types=[
        pltpu.VMEM((NCHUNK, CHUNK), jnp.int32),
        pltpu.VMEM((NCHUNK, CHUNK), jnp.int32),
        pltpu.VMEM((NCHUNK, CHUNK), jnp.int32),
        pltpu.VMEM((NCHUNK, CHUNK), jnp.int32),
        pltpu.VMEM((CHUNK, PACK * D), jnp.float32),
        pltpu.VMEM((CHUNK, PACK * D), jnp.float32),
        pltpu.VMEM((BPW,), jnp.float32),
        pltpu.SemaphoreType.DMA,
    ],
)
def _cf_kernel(user_hbm, item_hbm, upack_hbm, ipack_hbm, out_hbm,
               uidx_v, iidx_v, uq_v, iq_v, ubuf_v, ibuf_v, out_v, sem):
    wid = lax.axis_index("s") * NC + lax.axis_index("c")
    base = wid * BPW

    pltpu.sync_copy(user_hbm.at[pl.ds(wid * NCHUNK, NCHUNK)], uidx_v)
    pltpu.sync_copy(item_hbm.at[pl.ds(wid * NCHUNK, NCHUNK)], iidx_v)

    def _q(r):
        return (lax.shift_left(lax.shift_right_logical(r, 9), 7)
                + (r & 127))

    def qbody(v, carry):
        j = v // (CHUNK // L)
        col = (v % (CHUNK // L)) * L
        uq_v[j, pl.ds(col, L)] = _q(uidx_v[j, pl.ds(col, L)])
        iq_v[j, pl.ds(col, L)] = _q(iidx_v[j, pl.ds(col, L)])
        return carry

    lax.fori_loop(0, NCHUNK * CHUNK // L, qbody, 0)

    lanes = lax.iota(jnp.int32, L)

    def chunk_body(j, carry):
        cu = pltpu.async_copy(upack_hbm.at[uq_v.at[j]], ubuf_v, sem)
        ci = pltpu.async_copy(ipack_hbm.at[iq_v.at[j]], ibuf_v, sem)
        cu.wait()
        ci.wait()
        for k in range(CHUNK // L):
            pvec = k * L + lanes
            ubase = lax.shift_left(
                lax.shift_right_logical(uidx_v[j, pl.ds(k * L, L)], 7) & 3, 5)
            ibase = lax.shift_left(
                lax.shift_right_logical(iidx_v[j, pl.ds(k * L, L)], 7) & 3, 5)
            acc = jnp.zeros((L,), jnp.float32)
            for d in range(D):
                u = plsc.load_gather(ubuf_v, [pvec, ubase + d])
                it = plsc.load_gather(ibuf_v, [pvec, ibase + d])
                acc = acc + u * it
            out_v[pl.ds(j * CHUNK + k * L, L)] = acc
        return carry

    lax.fori_loop(0, NCHUNK, chunk_body, 0)

    pltpu.sync_copy(out_v, out_hbm.at[pl.ds(base, BPW)])


def kernel(user, item, user_table, item_table):
    utabT = user_table.T
    itabT = item_table.T
    upack = _repack_tc(utabT)
    ipack = _repack_sc(itabT, itabT[:, V - 64:])
    user2 = user.reshape(NW * NCHUNK, CHUNK)
    item2 = item.reshape(NW * NCHUNK, CHUNK)
    return _cf_kernel(user2, item2, upack, ipack)


# speedup vs baseline: 3.1800x; 1.1200x over previous
import functools

import jax
import jax.numpy as jnp
from jax import lax
from jax.experimental import pallas as pl
from jax.experimental.pallas import tpu as pltpu
from jax.experimental.pallas import tpu_sc as plsc

B = 16384
V = 1000000
D = 32
L = 16
NC = 2
NS = 16
NW = NC * NS
BPW = B // NW
CHUNK = 128
NCHUNK = BPW // CHUNK
PACK = 128 // D
_NGRP = (V + 511) // 512
_GPW = (_NGRP + NW - 1) // NW
NPROWS = _NGRP * 128


RBLK = 2048
_GRID = (V + RBLK - 1) // RBLK


def _repack_tc_body(t_ref, o_ref):
    t = t_ref[...]
    tt = lax.dot_general(t, jnp.eye(D, dtype=jnp.float32),
                         (((0,), (0,)), ((), ())),
                         preferred_element_type=jnp.float32)
    for g in range(RBLK // 512):
        for s in range(PACK):
            lo = 512 * g + 128 * s
            o_ref[pl.ds(128 * g, 128), pl.ds(D * s, D)] = tt[lo:lo + 128, :]


_repack_tc = pl.pallas_call(
    _repack_tc_body,
    grid=(_GRID,),
    in_specs=[pl.BlockSpec((D, RBLK), lambda i: (0, i))],
    out_specs=pl.BlockSpec((RBLK // PACK, PACK * D), lambda i: (i, 0)),
    out_shape=jax.ShapeDtypeStruct((_GRID * (RBLK // PACK), PACK * D),
                                   jnp.float32),
    compiler_params=pltpu.CompilerParams(fuse_transposed_lhs_in_matmul=True),
)


_mesh_rp = plsc.VectorSubcoreMesh(core_axis_name="c", subcore_axis_name="s")


@functools.partial(
    pl.kernel,
    out_type=jax.ShapeDtypeStruct((NPROWS, PACK * D), jnp.float32),
    mesh=_mesh_rp,
    compiler_params=pltpu.CompilerParams(
        needs_layout_passes=False, use_tc_tiling_on_sc=True),
    scratch_types=[
        pltpu.VMEM((D, 512), jnp.float32),
        pltpu.VMEM((D, 512), jnp.float32),
        pltpu.VMEM((128, PACK * D), jnp.float32),
        pltpu.VMEM((D, 64), jnp.float32),
        pltpu.SemaphoreType.DMA,
    ],
)
def _repack_sc(tabT_hbm, tail_hbm, out_hbm, win_a, win_b, blk_v, tl_v, sem):
    wid = lax.axis_index("s") * NC + lax.axis_index("c")
    base = wid * _GPW
    n_own = jnp.minimum(_GPW, jnp.maximum((_NGRP - 1) - base, 0))
    lanes = lax.iota(jnp.int32, L)

    def _fire(i, win):
        pltpu.async_copy(tabT_hbm.at[:, pl.ds(i * 512, 512)], win, sem)

    def _wait(i, win):
        pltpu.make_async_copy(
            tabT_hbm.at[:, pl.ds(i * 512, 512)], win, sem).wait()

    def _transpose(win_v, i):
        def diag_body(j, carry2):
            perm = (lanes + j) & (L - 1)
            for d0 in range(0, D, L):
                dv = d0 + lanes
                for s in range(PACK):
                    for x0 in range(0, 128, L):
                        v = plsc.load_gather(
                            win_v, [dv, (s * 128 + x0) + perm])
                        plsc.store_scatter(
                            blk_v, [x0 + perm, (s * D + d0) + lanes], v)
            return carry2

        lax.fori_loop(0, L, diag_body, 0)
        pltpu.sync_copy(blk_v, out_hbm.at[pl.ds(i * 128, 128)])

    @pl.when(n_own > 0)
    def _prologue():
        _fire(base, win_a)

    def pair_body(p, carry):
        i0 = base + 2 * p
        i1 = i0 + 1

        @pl.when(2 * p < n_own)
        def _even():
            _wait(i0, win_a)

            @pl.when(2 * p + 1 < n_own)
            def _fire_b():
                _fire(i1, win_b)

            _transpose(win_a, i0)

        @pl.when(2 * p + 1 < n_own)
        def _odd():
            _wait(i1, win_b)

            @pl.when(2 * p + 2 < n_own)
            def _fire_a():
                _fire(i0 + 2, win_a)

            _transpose(win_b, i1)

        return carry

    lax.fori_loop(0, (_GPW + 1) // 2, pair_body, 0)

    @pl.when(wid == NW - 1)
    def _tail():
        pltpu.sync_copy(tail_hbm, tl_v)
        for d in range(D):
            col = jnp.full((L,), d, jnp.int32)
            for k in range(64 // L):
                xv = k * L + lanes
                v = tl_v[d, pl.ds(k * L, L)]
                plsc.store_scatter(blk_v, [xv, col], v)
        pltpu.sync_copy(blk_v, out_hbm.at[pl.ds((_NGRP - 1) * 128, 128)])


_mesh = plsc.VectorSubcoreMesh(core_axis_name="c", subcore_axis_name="s")


@functools.partial(
    pl.kernel,
    out_type=jax.ShapeDtypeStruct((B,), jnp.float32),
    mesh=_mesh,
    compiler_params=pltpu.CompilerParams(
        needs_layout_passes=False, use_tc_tiling_on_sc=False),
    scratch_types=[
        pltpu.VMEM((NCHUNK, CHUNK), jnp.int32),
        pltpu.VMEM((NCHUNK, CHUNK), jnp.int32),
        pltpu.VMEM((NCHUNK, CHUNK), jnp.int32),
        pltpu.VMEM((NCHUNK, CHUNK), jnp.int32),
        pltpu.VMEM((CHUNK, PACK * D), jnp.float32),
        pltpu.VMEM((CHUNK, PACK * D), jnp.float32),
        pltpu.VMEM((BPW,), jnp.float32),
        pltpu.SemaphoreType.DMA,
    ],
)
def _cf_kernel(user_hbm, item_hbm, upack_hbm, ipack_hbm, out_hbm,
               uidx_v, iidx_v, uq_v, iq_v, ubuf_v, ibuf_v, out_v, sem):
    wid = lax.axis_index("s") * NC + lax.axis_index("c")
    base = wid * BPW

    pltpu.sync_copy(user_hbm.at[pl.ds(wid * NCHUNK, NCHUNK)], uidx_v)
    pltpu.sync_copy(item_hbm.at[pl.ds(wid * NCHUNK, NCHUNK)], iidx_v)

    def _q(r):
        return (lax.shift_left(lax.shift_right_logical(r, 9), 7)
                + (r & 127))

    def qbody(v, carry):
        j = v // (CHUNK // L)
        col = (v % (CHUNK // L)) * L
        uq_v[j, pl.ds(col, L)] = _q(uidx_v[j, pl.ds(col, L)])
        iq_v[j, pl.ds(col, L)] = _q(iidx_v[j, pl.ds(col, L)])
        return carry

    lax.fori_loop(0, NCHUNK * CHUNK // L, qbody, 0)

    lanes = lax.iota(jnp.int32, L)

    def chunk_body(j, carry):
        cu = pltpu.async_copy(upack_hbm.at[uq_v.at[j]], ubuf_v, sem)
        ci = pltpu.async_copy(ipack_hbm.at[iq_v.at[j]], ibuf_v, sem)
        cu.wait()
        ci.wait()
        for k in range(CHUNK // L):
            pvec = k * L + lanes
            ubase = lax.shift_left(
                lax.shift_right_logical(uidx_v[j, pl.ds(k * L, L)], 7) & 3, 5)
            ibase = lax.shift_left(
                lax.shift_right_logical(iidx_v[j, pl.ds(k * L, L)], 7) & 3, 5)
            acc = jnp.zeros((L,), jnp.float32)
            for d in range(D):
                u = plsc.load_gather(ubuf_v, [pvec, ubase + d])
                it = plsc.load_gather(ibuf_v, [pvec, ibase + d])
                acc = acc + u * it
            out_v[pl.ds(j * CHUNK + k * L, L)] = acc
        return carry

    lax.fori_loop(0, NCHUNK, chunk_body, 0)

    pltpu.sync_copy(out_v, out_hbm.at[pl.ds(base, BPW)])


def kernel(user, item, user_table, item_table):
    utabT = user_table.T
    itabT = item_table.T
    upack = _repack_tc(utabT)
    ipack = _repack_sc(itabT, itabT[:, V - 64:])
    user2 = user.reshape(NW * NCHUNK, CHUNK)
    item2 = item.reshape(NW * NCHUNK, CHUNK)
    return _cf_kernel(user2, item2, upack, ipack)
